# baseline (speedup 1.0000x reference)
import functools

import jax
import jax.numpy as jnp
from jax import lax
from jax.experimental import pallas as pl
from jax.experimental.pallas import tpu as pltpu
from jax.experimental.pallas import tpu_sc as plsc

_S = 7
_B = 2
_C = 20
_LC = 5.0
_LN = 0.5
_BATCH = 4096
_CH = 5 * _B + _C
_CELLS = _BATCH * _S * _S
_NW = 32
_CELLS_W = _CELLS // _NW
_NCHUNK = 8
_CHUNK_CELLS = _CELLS_W // _NCHUNK
_CHUNK_WORDS = _CHUNK_CELLS * _CH
_GROUPS = _CHUNK_CELLS // 16
_LANES = 16


def _fsqrt(x):
  x = jnp.maximum(x, jnp.float32(1e-30))
  i = plsc.bitcast(x, jnp.int32)
  y = plsc.bitcast(jnp.int32(0x5F3759DF) - (i >> 1), jnp.float32)
  half_x = jnp.float32(0.5) * x
  y = y * (jnp.float32(1.5) - half_x * y * y)
  y = y * (jnp.float32(1.5) - half_x * y * y)
  y = y * (jnp.float32(1.5) - half_x * y * y)
  return x * y


def _cell_group(pbuf, tbuf, g, iota_ch):
  base = g * (_LANES * _CH)
  idxs = [iota_ch + (base + c) for c in range(_CH)]
  p = [plsc.load_gather(pbuf, [ix]) for ix in idxs]
  t = [plsc.load_gather(tbuf, [ix]) for ix in idxs]

  one = jnp.float32(1.0)
  zero = jnp.float32(0.0)
  objf = jnp.where(t[4] > zero, one, zero)
  noobjf = jnp.where(t[4] == zero, one, zero)

  cls = None
  for c in range(5 * _B, _CH):
    d = p[c] - t[c]
    cls = d * d if cls is None else cls + d * d

  d4 = p[4] - t[4]
  d9 = p[9] - t[9]
  noobj = d4 * d4 + d9 * d9

  inv_s = jnp.float32(1.0 / _S)
  half = jnp.float32(0.5)
  tcx = t[0] * inv_s
  tcy = t[1] * inv_s
  thw = half * t[2]
  thh = half * t[3]
  t1x = tcx - thw
  t1y = tcy - thh
  t2x = tcx + thw
  t2y = tcy + thh
  area_t = (t2x - t1x) * (t2y - t1y)
  ious = []
  for b in range(_B):
    o = 5 * b
    pcx = p[o] * inv_s
    pcy = p[o + 1] * inv_s
    phw = half * p[o + 2]
    phh = half * p[o + 3]
    p1x = pcx - phw
    p1y = pcy - phh
    p2x = pcx + phw
    p2y = pcy + phh
    ltx = jnp.maximum(p1x, t1x)
    lty = jnp.maximum(p1y, t1y)
    rbx = jnp.minimum(p2x, t2x)
    rby = jnp.minimum(p2y, t2y)
    wx = jnp.maximum(rbx - ltx, zero)
    wy = jnp.maximum(rby - lty, zero)
    inter = wx * wy
    area_p = (p2x - p1x) * (p2y - p1y)
    ious.append(inter / (area_p + area_t - inter))

  sel = ious[1] > ious[0]
  max_iou = jnp.maximum(ious[0], ious[1])
  pr = [jnp.where(sel, p[5 + k], p[k]) for k in range(5)]
  tr = [jnp.where(sel, t[5 + k], t[k]) for k in range(5)]

  dx = pr[0] - tr[0]
  dy = pr[1] - tr[1]
  dxy = dx * dx + dy * dy
  sw = _fsqrt(pr[2]) - _fsqrt(tr[2])
  sh = _fsqrt(pr[3]) - _fsqrt(tr[3])
  dwh = sw * sw + sh * sh
  do = pr[4] - max_iou
  dobj = do * do

  return objf * (jnp.float32(_LC) * (dxy + dwh) + dobj + cls) + \
      jnp.float32(_LN) * (noobjf * noobj)


_mesh = plsc.VectorSubcoreMesh(core_axis_name="c", subcore_axis_name="s")


@functools.partial(
    pl.kernel,
    out_type=jax.ShapeDtypeStruct((_NW, _LANES), jnp.float32),
    mesh=_mesh,
    scratch_types=[
        pltpu.VMEM((_CHUNK_WORDS,), jnp.float32),
        pltpu.VMEM((_CHUNK_WORDS,), jnp.float32),
        pltpu.VMEM((_CHUNK_WORDS,), jnp.float32),
        pltpu.VMEM((_CHUNK_WORDS,), jnp.float32),
        pltpu.VMEM((_LANES,), jnp.float32),
        pltpu.SemaphoreType.DMA,
        pltpu.SemaphoreType.DMA,
    ],
    compiler_params=pltpu.CompilerParams(needs_layout_passes=False),
)
def _sc_loss(pred_hbm, tgt_hbm, out_hbm, p_a, p_b, t_a, t_b, accbuf,
             sem_a, sem_b):
  wid = lax.axis_index("s") * 2 + lax.axis_index("c")
  wbase = wid * (_CELLS_W * _CH)
  iota_ch = lax.iota(jnp.int32, _LANES) * _CH
  pbufs = (p_a, p_b)
  tbufs = (t_a, t_b)
  sems = (sem_a, sem_b)

  def start(ci):
    off = pl.multiple_of(wbase + ci * _CHUNK_WORDS, 8)
    buf = ci % 2
    h1 = pltpu.async_copy(pred_hbm.at[pl.ds(off, _CHUNK_WORDS)],
                          pbufs[buf], sems[buf])
    h2 = pltpu.async_copy(tgt_hbm.at[pl.ds(off, _CHUNK_WORDS)],
                          tbufs[buf], sems[buf])
    return h1, h2

  pending = start(0)
  acc = jnp.zeros((_LANES,), jnp.float32)
  for ci in range(_NCHUNK):
    pending[0].wait()
    pending[1].wait()
    if ci + 1 < _NCHUNK:
      nxt = start(ci + 1)
    buf = ci % 2

    def body(g, a, _pb=pbufs[buf], _tb=tbufs[buf]):
      return a + _cell_group(_pb, _tb, g, iota_ch)

    acc = lax.fori_loop(0, _GROUPS, body, acc)
    if ci + 1 < _NCHUNK:
      pending = nxt

  accbuf[...] = acc
  pltpu.sync_copy(accbuf, out_hbm.at[wid])


def _finish_body(x_ref, o_ref):
  o_ref[0, 0] = jnp.sum(x_ref[...]) * jnp.float32(1.0 / _BATCH)


_finish = pl.pallas_call(
    _finish_body,
    out_shape=jax.ShapeDtypeStruct((1, 1), jnp.float32),
    out_specs=pl.BlockSpec(memory_space=pltpu.SMEM),
)


def kernel(pred_tensor, target_tensor):
  parts = _sc_loss(pred_tensor.reshape(-1), target_tensor.reshape(-1))
  return _finish(parts)[0, 0]


# speedup vs baseline: 3.5217x; 3.5217x over previous
import functools

import jax
import jax.numpy as jnp
from jax import lax
from jax.experimental import pallas as pl
from jax.experimental.pallas import tpu as pltpu
from jax.experimental.pallas import tpu_sc as plsc

_S = 7
_B = 2
_C = 20
_LC = 5.0
_LN = 0.5
_BATCH = 4096
_CH = 5 * _B + _C
_SLABS = _S * _S
_NW = 32
_BPW = _BATCH // _NW
_LANES = 16
_KG = _BPW // _LANES


def _fsqrt(x):
  x = jnp.maximum(x, jnp.float32(1e-30))
  i = plsc.bitcast(x, jnp.int32)
  y = plsc.bitcast(jnp.int32(0x5F3759DF) - (i >> 1), jnp.float32)
  half_x = jnp.float32(0.5) * x
  y = y * (jnp.float32(1.5) - half_x * y * y)
  y = y * (jnp.float32(1.5) - half_x * y * y)
  y = y * (jnp.float32(1.5) - half_x * y * y)
  return x * y


def _lane_group(pbuf, tbuf, k):
  sl = pl.ds(k * _LANES, _LANES)
  p = [pbuf[c, sl] for c in range(_CH)]
  t = [tbuf[c, sl] for c in range(_CH)]

  one = jnp.float32(1.0)
  zero = jnp.float32(0.0)
  objf = jnp.where(t[4] > zero, one, zero)
  noobjf = jnp.where(t[4] == zero, one, zero)

  cls = None
  for c in range(5 * _B, _CH):
    d = p[c] - t[c]
    cls = d * d if cls is None else cls + d * d

  d4 = p[4] - t[4]
  d9 = p[9] - t[9]
  noobj = d4 * d4 + d9 * d9

  inv_s = jnp.float32(1.0 / _S)
  half = jnp.float32(0.5)
  tcx = t[0] * inv_s
  tcy = t[1] * inv_s
  thw = half * t[2]
  thh = half * t[3]
  t1x = tcx - thw
  t1y = tcy - thh
  t2x = tcx + thw
  t2y = tcy + thh
  area_t = (t2x - t1x) * (t2y - t1y)
  ious = []
  for b in range(_B):
    o = 5 * b
    pcx = p[o] * inv_s
    pcy = p[o + 1] * inv_s
    phw = half * p[o + 2]
    phh = half * p[o + 3]
    p1x = pcx - phw
    p1y = pcy - phh
    p2x = pcx + phw
    p2y = pcy + phh
    ltx = jnp.maximum(p1x, t1x)
    lty = jnp.maximum(p1y, t1y)
    rbx = jnp.minimum(p2x, t2x)
    rby = jnp.minimum(p2y, t2y)
    wx = jnp.maximum(rbx - ltx, zero)
    wy = jnp.maximum(rby - lty, zero)
    inter = wx * wy
    area_p = (p2x - p1x) * (p2y - p1y)
    ious.append(inter / (area_p + area_t - inter))

  sel = ious[1] > ious[0]
  max_iou = jnp.maximum(ious[0], ious[1])
  pr = [jnp.where(sel, p[5 + k_], p[k_]) for k_ in range(5)]
  tr = [jnp.where(sel, t[5 + k_], t[k_]) for k_ in range(5)]

  dx = pr[0] - tr[0]
  dy = pr[1] - tr[1]
  dxy = dx * dx + dy * dy
  sw = _fsqrt(pr[2]) - _fsqrt(tr[2])
  sh = _fsqrt(pr[3]) - _fsqrt(tr[3])
  dwh = sw * sw + sh * sh
  do = pr[4] - max_iou
  dobj = do * do

  return objf * (jnp.float32(_LC) * (dxy + dwh) + dobj + cls) + \
      jnp.float32(_LN) * (noobjf * noobj)


def _slab(pbuf, tbuf, acc):
  for k in range(_KG):
    acc = acc + _lane_group(pbuf, tbuf, k)
  return acc


_mesh = plsc.VectorSubcoreMesh(core_axis_name="c", subcore_axis_name="s")


@functools.partial(
    pl.kernel,
    out_type=jax.ShapeDtypeStruct((_NW, _LANES), jnp.float32),
    mesh=_mesh,
    scratch_types=[
        pltpu.VMEM((_CH, _BPW), jnp.float32),
        pltpu.VMEM((_CH, _BPW), jnp.float32),
        pltpu.VMEM((_CH, _BPW), jnp.float32),
        pltpu.VMEM((_CH, _BPW), jnp.float32),
        pltpu.VMEM((_LANES,), jnp.float32),
        pltpu.SemaphoreType.DMA,
        pltpu.SemaphoreType.DMA,
    ],
    compiler_params=pltpu.CompilerParams(use_tc_tiling_on_sc=True,
                                         needs_layout_passes=False),
)
def _sc_loss(pred_hbm, tgt_hbm, out_hbm, p_a, p_b, t_a, t_b, accbuf,
             sem_a, sem_b):
  wid = lax.axis_index("s") * 2 + lax.axis_index("c")
  boff = pl.multiple_of(wid * _BPW, _BPW)
  pbufs = (p_a, p_b)
  tbufs = (t_a, t_b)
  sems = (sem_a, sem_b)

  def start(s, buf):
    pltpu.async_copy(pred_hbm.at[s, :, pl.ds(boff, _BPW)],
                     pbufs[buf], sems[buf])
    pltpu.async_copy(tgt_hbm.at[s, :, pl.ds(boff, _BPW)],
                     tbufs[buf], sems[buf])

  def wait(s, buf):
    pltpu.make_async_copy(pred_hbm.at[s, :, pl.ds(boff, _BPW)],
                          pbufs[buf], sems[buf]).wait()
    pltpu.make_async_copy(tgt_hbm.at[s, :, pl.ds(boff, _BPW)],
                          tbufs[buf], sems[buf]).wait()

  start(0, 0)

  def pair_body(m, acc):
    s0 = 2 * m
    wait(s0, 0)
    start(s0 + 1, 1)
    acc = _slab(pbufs[0], tbufs[0], acc)
    wait(s0 + 1, 1)
    start(s0 + 2, 0)
    acc = _slab(pbufs[1], tbufs[1], acc)
    return acc

  acc = lax.fori_loop(0, (_SLABS - 1) // 2, pair_body,
                      jnp.zeros((_LANES,), jnp.float32))
  wait(_SLABS - 1, 0)
  acc = _slab(pbufs[0], tbufs[0], acc)

  accbuf[...] = acc
  pltpu.sync_copy(accbuf, out_hbm.at[wid])


def _finish_body(x_ref, o_ref):
  o_ref[0, 0] = jnp.sum(x_ref[...]) * jnp.float32(1.0 / _BATCH)


_finish = pl.pallas_call(
    _finish_body,
    out_shape=jax.ShapeDtypeStruct((1, 1), jnp.float32),
    out_specs=pl.BlockSpec(memory_space=pltpu.SMEM),
)


def kernel(pred_tensor, target_tensor):
  pv = jnp.transpose(pred_tensor, (1, 2, 3, 0)).reshape(_SLABS, _CH, _BATCH)
  tv = jnp.transpose(target_tensor, (1, 2, 3, 0)).reshape(_SLABS, _CH, _BATCH)
  parts = _sc_loss(pv, tv)
  return _finish(parts)[0, 0]
